# Initial kernel scaffold; baseline (speedup 1.0000x reference)
import jax
import jax.numpy as jnp
from jax.experimental import pallas as pl


def kernel(flat_h, flat_idx, flat_gate, gate_weight, up_weight, down_weight):
    raise NotImplementedError("write your pallas kernel here")


# speedup vs baseline: 4.1836x; 4.1836x over previous
import functools

import jax
import jax.numpy as jnp
from jax import lax
from jax.experimental import pallas as pl
from jax.experimental.pallas import tpu as pltpu
from jax.experimental.pallas import tpu_sc as plsc

M = 8192
HIDDEN = 2048
INTER = 1024
E = 16
TOPK = 2
R = M * TOPK
BR = 256
NB = R // BR + E
P = NB * BR
CLIP_LO, CLIP_HI = -10.0, 10.0
CLIP_GATE = 10.0

NW = 32
ROWS_W = P // NW
GCH = 16
NCH = ROWS_W // GCH
TOK_W = M // NW
TCH = TOK_W // GCH


def _sc_gather_rows(table, idx):
    mesh = plsc.VectorSubcoreMesh(core_axis_name="c", subcore_axis_name="s")

    @functools.partial(
        pl.kernel,
        out_type=jax.ShapeDtypeStruct((P, HIDDEN), jnp.float32),
        mesh=mesh,
        scratch_types=[
            pltpu.VMEM((NCH, GCH), jnp.int32),
            pltpu.VMEM((GCH, HIDDEN), jnp.float32),
            pltpu.SemaphoreType.DMA,
        ],
    )
    def k(table_hbm, idx_hbm, out_hbm, idx_v, buf, sem):
        wid = lax.axis_index("s") * 2 + lax.axis_index("c")
        pltpu.sync_copy(idx_hbm.at[wid], idx_v)
        base = wid * ROWS_W

        def body(c, carry):
            pltpu.async_copy(table_hbm.at[idx_v.at[c]], buf, sem).wait()
            pltpu.sync_copy(buf, out_hbm.at[pl.ds(base + c * GCH, GCH)])
            return carry

        lax.fori_loop(0, NCH, body, 0)

    return k(table, idx)


def _sc_combine_gather(y, idx_ab):
    mesh = plsc.VectorSubcoreMesh(core_axis_name="c", subcore_axis_name="s")

    @functools.partial(
        pl.kernel,
        out_type=(
            jax.ShapeDtypeStruct((M, HIDDEN), jnp.float32),
            jax.ShapeDtypeStruct((M, HIDDEN), jnp.float32),
        ),
        mesh=mesh,
        scratch_types=[
            pltpu.VMEM((2, TCH, GCH), jnp.int32),
            pltpu.VMEM((GCH, HIDDEN), jnp.float32),
            pltpu.SemaphoreType.DMA,
        ],
    )
    def k(y_hbm, idx_hbm, a_hbm, b_hbm, idx_v, buf, sem):
        wid = lax.axis_index("s") * 2 + lax.axis_index("c")
        pltpu.sync_copy(idx_hbm.at[wid], idx_v)
        base = wid * TOK_W

        def body(c, carry):
            pltpu.async_copy(y_hbm.at[idx_v.at[0, c]], buf, sem).wait()
            pltpu.sync_copy(buf, a_hbm.at[pl.ds(base + c * GCH, GCH)])
            pltpu.async_copy(y_hbm.at[idx_v.at[1, c]], buf, sem).wait()
            pltpu.sync_copy(buf, b_hbm.at[pl.ds(base + c * GCH, GCH)])
            return carry

        lax.fori_loop(0, TCH, body, 0)

    return k(y, idx_ab)


def _tc_grouped_gemm(x, gw, uw, dw, gate_col, be):
    def body(be_ref, x_ref, gw_ref, uw_ref, dw_ref, g_ref, y_ref):
        x = x_ref[...].astype(jnp.bfloat16)
        dn = (((1,), (1,)), ((), ()))
        go = lax.dot_general(x, gw_ref[0], dn, preferred_element_type=jnp.float32)
        uo = lax.dot_general(x, uw_ref[0], dn, preferred_element_type=jnp.float32)
        go = jnp.minimum(go, CLIP_GATE)
        uo = jnp.clip(uo, CLIP_LO, CLIP_HI)
        h = (go * jax.nn.sigmoid(go)) * uo * g_ref[...]
        y_ref[...] = lax.dot_general(
            h.astype(jnp.bfloat16), dw_ref[0], dn, preferred_element_type=jnp.float32)

    grid_spec = pltpu.PrefetchScalarGridSpec(
        num_scalar_prefetch=1,
        grid=(NB,),
        in_specs=[
            pl.BlockSpec((BR, HIDDEN), lambda i, be: (i, 0)),
            pl.BlockSpec((1, INTER, HIDDEN), lambda i, be: (be[i], 0, 0)),
            pl.BlockSpec((1, INTER, HIDDEN), lambda i, be: (be[i], 0, 0)),
            pl.BlockSpec((1, HIDDEN, INTER), lambda i, be: (be[i], 0, 0)),
            pl.BlockSpec((BR, 1), lambda i, be: (i, 0)),
        ],
        out_specs=pl.BlockSpec((BR, HIDDEN), lambda i, be: (i, 0)),
    )
    return pl.pallas_call(
        body,
        grid_spec=grid_spec,
        out_shape=jax.ShapeDtypeStruct((P, HIDDEN), jnp.float32),
        compiler_params=pltpu.CompilerParams(dimension_semantics=("arbitrary",)),
    )(be, x, gw, uw, dw, gate_col)


def _tc_pair_add(a, b):
    def body(a_ref, b_ref, o_ref):
        o_ref[...] = a_ref[...] + b_ref[...]

    blk = 512
    return pl.pallas_call(
        body,
        grid=(M // blk,),
        in_specs=[pl.BlockSpec((blk, HIDDEN), lambda i: (i, 0))] * 2,
        out_specs=pl.BlockSpec((blk, HIDDEN), lambda i: (i, 0)),
        out_shape=jax.ShapeDtypeStruct((M, HIDDEN), jnp.float32),
    )(a, b)


def _dispatch_plan(flat_idx, flat_gate):
    i32 = jnp.int32
    expert_id = flat_idx.reshape(-1).astype(i32)
    gate = flat_gate.reshape(-1)
    perm = jnp.argsort(expert_id, stable=True).astype(i32)
    e_s = expert_id[perm]
    counts = jnp.bincount(expert_id, length=E).astype(i32)
    offsets = jnp.cumsum(counts) - counts
    blocks_per_e = (counts + BR - 1) // BR
    cum_blocks = jnp.cumsum(blocks_per_e)
    pad_offset = (cum_blocks - blocks_per_e) * BR
    s_ar = jnp.arange(R, dtype=i32)
    q_of_s = pad_offset[e_s] + (s_ar - offsets[e_s])
    src_token = jnp.zeros(P, i32).at[q_of_s].set(perm // TOPK)
    gate_padded = jnp.zeros(P, jnp.float32).at[q_of_s].set(gate[perm])
    be = jnp.clip(
        jnp.searchsorted(cum_blocks, jnp.arange(NB, dtype=i32), side="right"),
        0, E - 1).astype(i32)
    inv_padded = jnp.zeros(R, i32).at[perm].set(q_of_s)
    idx_a = inv_padded[0::2].reshape(NW, TCH, GCH)
    idx_b = inv_padded[1::2].reshape(NW, TCH, GCH)
    idx_ab = jnp.stack([idx_a, idx_b], axis=1)
    return src_token.reshape(NW, NCH, GCH), gate_padded.reshape(P, 1), be, idx_ab


def kernel(flat_h, flat_idx, flat_gate, gate_weight, up_weight, down_weight):
    src_token, gate_padded, be, idx_ab = _dispatch_plan(flat_idx, flat_gate)
    gathered = _sc_gather_rows(flat_h, src_token)
    gw = gate_weight.astype(jnp.bfloat16)
    uw = up_weight.astype(jnp.bfloat16)
    dw = down_weight.astype(jnp.bfloat16)
    y = _tc_grouped_gemm(gathered, gw, uw, dw, gate_padded, be)
    a, b = _sc_combine_gather(y, idx_ab)
    return _tc_pair_add(a, b)
